# baseline (speedup 1.0000x reference)
import jax
import jax.numpy as jnp
from jax.experimental import pallas as pl
from jax.experimental.pallas import tpu as pltpu

B, N, D_IN, D_H, D_OUT, K = 32, 512, 256, 256, 128, 16


def _dot(a, b):
    return jax.lax.dot_general(
        a, b, (((1,), (0,)), ((), ())),
        preferred_element_type=jnp.float32,
    )


PAIRS = 4
STEPS = B // PAIRS


def _gcntn_kernel(x1_ref, x2_ref, l1_ref, l2_ref, w1_ref, w2_ref, wtr_ref,
                  seg_ref, v1t_ref, v2t_ref, b_ref, wo_ref, out_ref, e_ref):
    b = pl.program_id(0)
    w1 = w1_ref[...]
    w2 = w2_ref[...]
    pool = jnp.full((1, N), 1.0 / N, dtype=jnp.float32)

    xs = [x1_ref[i] for i in range(PAIRS)] + [x2_ref[i] for i in range(PAIRS)]
    ls = [l1_ref[i] for i in range(PAIRS)] + [l2_ref[i] for i in range(PAIRS)]
    rows = ([b * PAIRS + i for i in range(PAIRS)]
            + [b * PAIRS + i + B for i in range(PAIRS)])

    bf = jnp.bfloat16
    xs = [x.astype(bf) for x in xs]
    ls = [l.astype(bf) for l in ls]
    w1 = w1.astype(bf)
    w2 = w2.astype(bf)
    xw = [_dot(x, w1).astype(bf) for x in xs]
    h = [jnp.maximum(_dot(l, v), 0.0).astype(bf) for l, v in zip(ls, xw)]
    hw = [_dot(v, w2).astype(bf) for v in h]
    h2 = [jnp.maximum(_dot(l, v), 0.0) for l, v in zip(ls, hw)]
    for row, v in zip(rows, h2):
        e_ref[pl.ds(row, 1), :] = _dot(pool, v)

    @pl.when(b == STEPS - 1)
    def _ntn():
        e1 = e_ref[0:B, :]
        e2 = e_ref[B:2 * B, :]
        t = _dot(e1, wtr_ref[...])
        bil = _dot(t * jnp.tile(e2, (1, K)), seg_ref[...])
        lin = _dot(e1, v1t_ref[...]) + _dot(e2, v2t_ref[...])
        ntn = jnp.maximum(bil + lin + b_ref[...], 0.0)
        out_ref[...] = _dot(ntn, wo_ref[...])


@jax.jit
def kernel(inputs_1, inputs_2, laplacians_1, laplacians_2, W1, W2, Wt, V,
           b_ntn, w_out):
    wt_r = jnp.transpose(Wt, (1, 0, 2)).reshape(D_OUT, K * D_OUT)
    seg = (jnp.arange(K * D_OUT)[:, None] // D_OUT
           == jnp.arange(K)[None, :]).astype(jnp.float32)
    v_t = V.T

    full = lambda *shape: pl.BlockSpec(shape, lambda b: (0,) * len(shape))
    batched = lambda *shape: pl.BlockSpec((PAIRS,) + shape,
                                          lambda b: (b,) + (0,) * len(shape))
    out = pl.pallas_call(
        _gcntn_kernel,
        grid=(STEPS,),
        in_specs=[
            batched(N, D_IN), batched(N, D_IN),
            batched(N, N), batched(N, N),
            full(D_IN, D_H), full(D_H, D_OUT),
            full(D_OUT, K * D_OUT), full(K * D_OUT, K),
            full(D_OUT, K), full(D_OUT, K),
            full(1, K), full(K, 1),
        ],
        out_specs=pl.BlockSpec((B, 1), lambda b: (0, 0)),
        out_shape=jax.ShapeDtypeStruct((B, 1), jnp.float32),
        scratch_shapes=[pltpu.VMEM((2 * B, D_OUT), jnp.float32)],
        compiler_params=pltpu.CompilerParams(
            dimension_semantics=("arbitrary",),
        ),
    )(inputs_1, inputs_2, laplacians_1, laplacians_2, W1, W2, wt_r, seg,
      v_t[:D_OUT], v_t[D_OUT:], b_ntn.reshape(1, K), w_out)
    return out[:, 0]


# speedup vs baseline: 1.0941x; 1.0941x over previous
import jax
import jax.numpy as jnp
from jax.experimental import pallas as pl
from jax.experimental.pallas import tpu as pltpu

B, N, D_IN, D_H, D_OUT, K = 32, 512, 256, 256, 128, 16

PAIRS = 4
STEPS = B // PAIRS


def _dot(a, b):
    return jax.lax.dot_general(
        a, b, (((1,), (0,)), ((), ())),
        preferred_element_type=jnp.float32,
    )


def _dot_t(a, b):
    return jax.lax.dot_general(
        a, b, (((1,), (1,)), ((), ())),
        preferred_element_type=jnp.float32,
    )


def _gcntn_kernel(x1_ref, x2_ref, l1_ref, l2_ref, w1_ref, w2_ref, wt2_ref,
                  v_ref, b_ref, wo_ref, out_ref, e_ref):
    b = pl.program_id(0)
    w1 = w1_ref[...]
    w2 = w2_ref[...]
    pool = jnp.full((1, N), 1.0 / N, dtype=jnp.float32)

    xs = [x1_ref[i] for i in range(PAIRS)] + [x2_ref[i] for i in range(PAIRS)]
    ls = [l1_ref[i] for i in range(PAIRS)] + [l2_ref[i] for i in range(PAIRS)]
    rows = ([b * PAIRS + i for i in range(PAIRS)]
            + [b * PAIRS + i + B for i in range(PAIRS)])

    xw = [_dot(x, w1) for x in xs]
    h = [jnp.maximum(_dot(l, v), 0.0) for l, v in zip(ls, xw)]
    hw = [_dot(v, w2) for v in h]
    h2 = [jnp.maximum(_dot(l, v), 0.0) for l, v in zip(ls, hw)]
    for row, v in zip(rows, h2):
        e_ref[pl.ds(row, 1), :] = _dot(pool, v)

    @pl.when(b == STEPS - 1)
    def _ntn():
        e1 = e_ref[0:B, :]
        e2 = e_ref[B:2 * B, :]
        t2 = _dot_t(e2, wt2_ref[...])
        seg = (jax.lax.broadcasted_iota(jnp.int32, (K * D_OUT, K), 0)
               // D_OUT
               == jax.lax.broadcasted_iota(jnp.int32, (K * D_OUT, K), 1)
               ).astype(jnp.float32)
        bil = _dot(t2 * jnp.tile(e1, (1, K)), seg)
        v = v_ref[...]
        lin = _dot_t(e1, v[:, :D_OUT]) + _dot_t(e2, v[:, D_OUT:])
        ntn = jnp.maximum(bil + lin + b_ref[...], 0.0)
        out_ref[...] = _dot(ntn, wo_ref[...])


@jax.jit
def kernel(inputs_1, inputs_2, laplacians_1, laplacians_2, W1, W2, Wt, V,
           b_ntn, w_out):
    full = lambda *shape: pl.BlockSpec(shape, lambda b: (0,) * len(shape))
    batched = lambda *shape: pl.BlockSpec((PAIRS,) + shape,
                                          lambda b: (b,) + (0,) * len(shape))
    out = pl.pallas_call(
        _gcntn_kernel,
        grid=(STEPS,),
        in_specs=[
            batched(N, D_IN), batched(N, D_IN),
            batched(N, N), batched(N, N),
            full(D_IN, D_H), full(D_H, D_OUT),
            full(K * D_OUT, D_OUT), full(K, 2 * D_OUT),
            full(1, K), full(K, 1),
        ],
        out_specs=pl.BlockSpec((B, 1), lambda b: (0, 0)),
        out_shape=jax.ShapeDtypeStruct((B, 1), jnp.float32),
        scratch_shapes=[pltpu.VMEM((2 * B, D_OUT), jnp.float32)],
        compiler_params=pltpu.CompilerParams(
            dimension_semantics=("arbitrary",),
        ),
    )(inputs_1, inputs_2, laplacians_1, laplacians_2, W1, W2,
      Wt.reshape(K * D_OUT, D_OUT), V, b_ntn.reshape(1, K), w_out)
    return out[:, 0]
